# Initial kernel scaffold; baseline (speedup 1.0000x reference)
import jax
import jax.numpy as jnp
from jax.experimental import pallas as pl


def kernel(x, edge_index, W1l, b1l, W1r, g1, be1, W2l, b2l, W2r, g2, be2, Wc1, bc1, Wc2, bc2):
    raise NotImplementedError("write your pallas kernel here")


# speedup vs baseline: 5.1416x; 5.1416x over previous
import functools

import jax
import jax.numpy as jnp
from jax import lax
from jax.experimental import pallas as pl
from jax.experimental.pallas import tpu as pltpu
from jax.experimental.pallas import tpu_sc as plsc

NN = 10000
EE = 320000
HH = 128

NC = 2
NS = 16
NW = NC * NS
EPW = EE // NW
CH = 128
NFULL = EPW // CH
REM = EPW - NFULL * CH
NPAD = 10240
ZR = NPAD // NS

BN_ = 400
NB = NN // BN_

_f32 = jnp.float32


def _mm_t(a, w):
  return lax.dot_general(a, w, (((1,), (1,)), ((), ())),
                         preferred_element_type=_f32)


def _make_seg(with_cnt):
  mesh = plsc.VectorSubcoreMesh(core_axis_name="c", subcore_axis_name="s",
                                num_cores=NC, num_subcores=NS)

  out_type = [jax.ShapeDtypeStruct((NC, NPAD, HH), _f32)]
  scratch = [
      pltpu.VMEM((CH,), jnp.int32),
      pltpu.VMEM((CH,), jnp.int32),
      pltpu.VMEM((CH, HH), _f32),
      pltpu.VMEM((REM,), jnp.int32),
      pltpu.VMEM((REM,), jnp.int32),
      pltpu.VMEM((REM, HH), _f32),
      pltpu.VMEM_SHARED((NPAD, HH), _f32),
      pltpu.SemaphoreType.DMA,
  ]
  if with_cnt:
    out_type.append(jax.ShapeDtypeStruct((NC, NPAD, HH), _f32))
    scratch += [
        pltpu.VMEM((REM, HH), _f32),
    ]

  def body(*refs):
    if with_cnt:
      (t, srcm, dstm, zseg, onesh,
       seg_o, cnt_o,
       sbuf, dbuf, rows, sbuf_r, dbuf_r, rows_r, acc, sem, onesr) = refs
    else:
      (t, srcm, dstm, zseg,
       seg_o,
       sbuf, dbuf, rows, sbuf_r, dbuf_r, rows_r, acc, sem) = refs

    c = lax.axis_index("c")
    s = lax.axis_index("s")
    base = (c * NS + s) * EPW

    pltpu.sync_copy(zseg, acc.at[pl.ds(s * ZR, ZR)])
    plsc.subcore_barrier()

    def step(i, carry):
      off = base + i * CH
      pltpu.sync_copy(srcm.at[pl.ds(off, CH)], sbuf)
      pltpu.sync_copy(dstm.at[pl.ds(off, CH)], dbuf)
      pltpu.async_copy(t.at[sbuf], rows, sem).wait()
      pltpu.sync_copy(rows, acc.at[dbuf], add=True)
      return carry

    lax.fori_loop(0, NFULL, step, 0)

    offr = base + NFULL * CH
    pltpu.sync_copy(srcm.at[pl.ds(offr, REM)], sbuf_r)
    pltpu.sync_copy(dstm.at[pl.ds(offr, REM)], dbuf_r)
    pltpu.async_copy(t.at[sbuf_r], rows_r, sem).wait()
    pltpu.sync_copy(rows_r, acc.at[dbuf_r], add=True)

    plsc.subcore_barrier()

    pltpu.sync_copy(acc.at[pl.ds(s * ZR, ZR)], seg_o.at[c, pl.ds(s * ZR, ZR)])

    if with_cnt:
      pltpu.sync_copy(zseg, acc.at[pl.ds(s * ZR, ZR)])
      pltpu.sync_copy(onesh, rows)
      pltpu.sync_copy(onesh.at[pl.ds(0, REM)], onesr)
      plsc.subcore_barrier()

      def cstep(i, carry):
        off = base + i * CH
        pltpu.sync_copy(dstm.at[pl.ds(off, CH)], dbuf)
        pltpu.sync_copy(rows, acc.at[dbuf], add=True)
        return carry

      lax.fori_loop(0, NFULL, cstep, 0)
      pltpu.sync_copy(dstm.at[pl.ds(offr, REM)], dbuf_r)
      pltpu.sync_copy(onesr, acc.at[dbuf_r], add=True)

      plsc.subcore_barrier()
      pltpu.sync_copy(acc.at[pl.ds(s * ZR, ZR)],
                      cnt_o.at[c, pl.ds(s * ZR, ZR)])

  return pl.kernel(body, out_type=tuple(out_type), mesh=mesh,
                   scratch_types=scratch)


@functools.cache
def _get_seg(with_cnt):
  return _make_seg(with_cnt)


def _pre_body(x_ref, wl_ref, wr_ref, b_ref, t_ref, r_ref):
  xb = x_ref[...]
  t_ref[...] = _mm_t(xb, wl_ref[...])
  r_ref[...] = _mm_t(xb, wr_ref[...]) + b_ref[...]


_pre = pl.pallas_call(
    _pre_body,
    grid=(NB,),
    in_specs=[
        pl.BlockSpec((BN_, HH), lambda i: (i, 0)),
        pl.BlockSpec((HH, HH), lambda i: (0, 0)),
        pl.BlockSpec((HH, HH), lambda i: (0, 0)),
        pl.BlockSpec((1, HH), lambda i: (0, 0)),
    ],
    out_specs=[
        pl.BlockSpec((BN_, HH), lambda i: (i, 0)),
        pl.BlockSpec((BN_, HH), lambda i: (i, 0)),
    ],
    out_shape=[jax.ShapeDtypeStruct((NN, HH), _f32)] * 2,
)


def _comb_body(p0_ref, p1_ref, c0_ref, c1_ref, r_ref, y_ref, st_ref):
  i = pl.program_id(0)
  cnt = jnp.maximum(c0_ref[...] + c1_ref[...], 1.0)
  y = (p0_ref[...] + p1_ref[...]) / cnt[:, 0:1] + r_ref[...]
  y_ref[...] = y

  @pl.when(i == 0)
  def _():
    st_ref[...] = jnp.zeros_like(st_ref)

  st_ref[0:1, :] += jnp.sum(y, axis=0, keepdims=True)
  st_ref[1:2, :] += jnp.sum(y * y, axis=0, keepdims=True)


_comb = pl.pallas_call(
    _comb_body,
    grid=(NB,),
    in_specs=[
        pl.BlockSpec((BN_, HH), lambda i: (i, 0)),
        pl.BlockSpec((BN_, HH), lambda i: (i, 0)),
        pl.BlockSpec((BN_, HH), lambda i: (i, 0)),
        pl.BlockSpec((BN_, HH), lambda i: (i, 0)),
        pl.BlockSpec((BN_, HH), lambda i: (i, 0)),
    ],
    out_specs=[
        pl.BlockSpec((BN_, HH), lambda i: (i, 0)),
        pl.BlockSpec((8, HH), lambda i: (0, 0)),
    ],
    out_shape=[
        jax.ShapeDtypeStruct((NN, HH), _f32),
        jax.ShapeDtypeStruct((8, HH), _f32),
    ],
)


def _bn_coeffs(st, g, be):
  m = st[0:1, :] * (1.0 / NN)
  v = st[1:2, :] * (1.0 / NN) - m * m
  sc = g / jnp.sqrt(v + 1e-5)
  sh = be - m * sc
  return sc, sh


def _apply_body(y_ref, st_ref, g_ref, be_ref, wl_ref, wr_ref, b_ref,
                t_ref, r_ref):
  sc, sh = _bn_coeffs(st_ref[...], g_ref[...], be_ref[...])
  h = jnp.maximum(y_ref[...] * sc + sh, 0.0)
  t_ref[...] = _mm_t(h, wl_ref[...])
  r_ref[...] = _mm_t(h, wr_ref[...]) + b_ref[...]


_apply = pl.pallas_call(
    _apply_body,
    grid=(NB,),
    in_specs=[
        pl.BlockSpec((BN_, HH), lambda i: (i, 0)),
        pl.BlockSpec((8, HH), lambda i: (0, 0)),
        pl.BlockSpec((1, HH), lambda i: (0, 0)),
        pl.BlockSpec((1, HH), lambda i: (0, 0)),
        pl.BlockSpec((HH, HH), lambda i: (0, 0)),
        pl.BlockSpec((HH, HH), lambda i: (0, 0)),
        pl.BlockSpec((1, HH), lambda i: (0, 0)),
    ],
    out_specs=[
        pl.BlockSpec((BN_, HH), lambda i: (i, 0)),
        pl.BlockSpec((BN_, HH), lambda i: (i, 0)),
    ],
    out_shape=[jax.ShapeDtypeStruct((NN, HH), _f32)] * 2,
)


def _fin_body(y_ref, st_ref, g_ref, be_ref, wc1_ref, bc1_ref, wc2_ref,
              bc2_ref, o_ref):
  sc, sh = _bn_coeffs(st_ref[...], g_ref[...], be_ref[...])
  h = jnp.maximum(y_ref[...] * sc + sh, 0.0)
  cmid = jnp.maximum(_mm_t(h, wc1_ref[...]) + bc1_ref[...], 0.0)
  o_ref[...] = _mm_t(cmid, wc2_ref[...]) + bc2_ref[...]


_fin = pl.pallas_call(
    _fin_body,
    grid=(NB,),
    in_specs=[
        pl.BlockSpec((BN_, HH), lambda i: (i, 0)),
        pl.BlockSpec((8, HH), lambda i: (0, 0)),
        pl.BlockSpec((1, HH), lambda i: (0, 0)),
        pl.BlockSpec((1, HH), lambda i: (0, 0)),
        pl.BlockSpec((HH // 2, HH), lambda i: (0, 0)),
        pl.BlockSpec((1, HH // 2), lambda i: (0, 0)),
        pl.BlockSpec((8, HH // 2), lambda i: (0, 0)),
        pl.BlockSpec((1, 8), lambda i: (0, 0)),
    ],
    out_specs=[pl.BlockSpec((BN_, 8), lambda i: (i, 0))],
    out_shape=[jax.ShapeDtypeStruct((NN, 8), _f32)],
)


def kernel(x, edge_index, W1l, b1l, W1r, g1, be1, W2l, b2l, W2r, g2, be2,
           Wc1, bc1, Wc2, bc2):
  src = edge_index[0]
  dst = edge_index[1]

  zseg = jnp.zeros((ZR, HH), _f32)
  onesh = jnp.ones((CH, HH), _f32)
  wc2p = jnp.zeros((8, HH // 2), _f32).at[:3, :].set(Wc2)
  bc2p = jnp.zeros((1, 8), _f32).at[0, :3].set(bc2)

  t1, r1 = _pre(x, W1l, W1r, b1l.reshape(1, HH))
  segp, cntp = _get_seg(True)(t1, src, dst, zseg, onesh)
  c0, c1 = cntp[0], cntp[1]
  y1, st1 = _comb(segp[0], segp[1], c0, c1, r1)
  t2, r2 = _apply(y1, st1, g1.reshape(1, HH), be1.reshape(1, HH),
                  W2l, W2r, b2l.reshape(1, HH))
  seg2p = _get_seg(False)(t2, src, dst, zseg)
  if isinstance(seg2p, (tuple, list)):
    seg2p = seg2p[0]
  y2, st2 = _comb(seg2p[0], seg2p[1], c0, c1, r2)
  (o,) = _fin(y2, st2, g2.reshape(1, HH), be2.reshape(1, HH),
              Wc1, bc1.reshape(1, HH // 2), wc2p, bc2p)
  return o[:, :3]
